# baseline (speedup 1.0000x reference)
import functools

import jax
import jax.numpy as jnp
from jax import lax
from jax.experimental import pallas as pl
from jax.experimental.pallas import tpu as pltpu

H = 12
D = 64
DIM = H * D
LR_BASE = 0.1
MOMENTUM = 0.9
SC = 256


def _gelu_exact(x):
    return x * 0.5 * (1.0 + lax.erf(x * 0.7071067811865476))


def _ln_rows(x, g, b, eps=1e-5):
    m = jnp.mean(x, axis=-1, keepdims=True)
    v = jnp.mean((x - m) ** 2, axis=-1, keepdims=True)
    return (x - m) * lax.rsqrt(v + eps) * g + b


def _proj_kernel(x_ref, wk_ref, wv_ref, wq_ref, ling_ref, linb_ref,
                 gseg_ref, gk_ref, bk_ref, gv_ref, bv_ref, gq_ref, bq_ref,
                 kt_ref, vt_ref, qt_ref):
    xn = _ln_rows(x_ref[0], ling_ref[...], linb_ref[...])
    gseg = gseg_ref[...]
    for w_ref, g_ref, b_ref, o_ref in (
        (wk_ref, gk_ref, bk_ref, kt_ref),
        (wv_ref, gv_ref, bv_ref, vt_ref),
        (wq_ref, gq_ref, bq_ref, qt_ref),
    ):
        p = jnp.dot(xn, w_ref[...], preferred_element_type=jnp.float32)
        m = jnp.dot(p, gseg, preferred_element_type=jnp.float32)
        mb = jnp.dot(m, gseg.T, preferred_element_type=jnp.float32) * float(D)
        d = p - mb
        v = jnp.dot(d * d, gseg, preferred_element_type=jnp.float32)
        vb = jnp.dot(v, gseg.T, preferred_element_type=jnp.float32) * float(D)
        o_ref[0] = d * lax.rsqrt(vb + 1e-5) * g_ref[...] + b_ref[...]


def _scan_kernel(kt_hbm, vt_hbm, qt_hbm, m0_ref, mom0_ref,
                 w1_ref, b1_ref, w2_ref, b2_ref, w3_ref, b3_ref,
                 blr_ref, bfg_ref,
                 o_hbm, mt_ref, momt_ref,
                 kbuf, vbuf, qbuf, obuf, insem, outsem):
    j = pl.program_id(0)

    @pl.when(j == 0)
    def _():
        mt_ref[...] = m0_ref[...]
        momt_ref[...] = mom0_ref[...]

    copies = []
    for ai, (src, dst) in enumerate(
            ((kt_hbm, kbuf), (vt_hbm, vbuf), (qt_hbm, qbuf))):
        for bi in range(4):
            cp = pltpu.make_async_copy(
                src.at[bi, pl.ds(j * SC, SC)],
                dst.at[:, bi * H:(bi + 1) * H, :],
                insem.at[ai * 4 + bi])
            cp.start()
            copies.append(cp)
    for cp in copies:
        cp.wait()

    w1 = w1_ref[...]
    b1 = b1_ref[...]
    w2 = w2_ref[...]
    b2 = b2_ref[...]
    w3 = w3_ref[...]
    b3 = b3_ref[...]
    blr = blr_ref[...]
    bfg = bfg_ref[...]

    def step(t, _):
        mt = mt_ref[...]
        mom = momt_ref[...]
        k_t = kbuf[t]
        v_t = vbuf[t]
        q_t = qbuf[t]

        ksub = k_t[:, :, None]
        out_t = jnp.sum(mt * q_t[:, :, None], axis=1)
        pred = jnp.sum(mt * ksub, axis=1)
        surprise = pred - v_t

        cat = jnp.concatenate([k_t, v_t, surprise], axis=1)
        h1 = _gelu_exact(jnp.dot(cat, w1, preferred_element_type=jnp.float32)
                         + b1)
        h2 = _gelu_exact(jnp.dot(h1, w2, preferred_element_type=jnp.float32)
                         + b2)
        mods = jax.nn.sigmoid(jnp.dot(h2, w3, preferred_element_type=jnp.float32)
                              + b3)
        eff_lr = blr * (0.5 + mods[:, :D])
        eff_fg = bfg * (0.5 + mods[:, D:])

        rnorm = 1.0 / (jnp.sqrt(jnp.sum(k_t * k_t, axis=1, keepdims=True)) + 1e-6)
        pred2 = pred * rnorm

        knsub = ksub * rnorm[:, :, None]
        mom2 = MOMENTUM * mom + knsub * surprise[:, None, :]
        w = eff_fg * pred2 + eff_lr * surprise
        elm = eff_lr * MOMENTUM
        mt_ref[...] = mt - knsub * w[:, None, :] \
                         - elm[:, None, :] * mom
        momt_ref[...] = mom2

        obuf[t] = out_t
        return ()

    lax.fori_loop(0, SC, step, (), unroll=2)

    ocopies = []
    for bi in range(4):
        cp = pltpu.make_async_copy(
            obuf.at[:, bi * H:(bi + 1) * H, :],
            o_hbm.at[bi, pl.ds(j * SC, SC)],
            outsem.at[bi])
        cp.start()
        ocopies.append(cp)
    for cp in ocopies:
        cp.wait()


def _out_kernel(x_ref, o_ref, wo_ref, gw_ref, gb_ref, ling_ref, linb_ref,
                y_ref):
    xn = _ln_rows(x_ref[0], ling_ref[...], linb_ref[...])
    out = jnp.dot(o_ref[0], wo_ref[...],
                  preferred_element_type=jnp.float32)
    gate = jax.nn.sigmoid(jnp.dot(xn, gw_ref[...],
                                  preferred_element_type=jnp.float32)
                          + gb_ref[...])
    y_ref[0] = gate * out


@jax.jit
def kernel(x, memory_state, memory_momentum, Wk, Wv, Wq, Wo,
           ln_in_g, ln_in_b, ln_k_g, ln_k_b, ln_v_g, ln_v_b, ln_q_g, ln_q_b,
           mm_w1, mm_b1, mm_w2, mm_b2, mm_w3, mm_b3,
           lr_scale, forget_scale, gate_w, gate_b):
    B, S, _ = x.shape
    C = S // SC
    f32 = jnp.float32

    ling = ln_in_g.reshape(1, DIM)
    linb = ln_in_b.reshape(1, DIM)
    gk, bk = jnp.tile(ln_k_g, H).reshape(1, DIM), jnp.tile(ln_k_b, H).reshape(1, DIM)
    gv, bv = jnp.tile(ln_v_g, H).reshape(1, DIM), jnp.tile(ln_v_b, H).reshape(1, DIM)
    gq, bq = jnp.tile(ln_q_g, H).reshape(1, DIM), jnp.tile(ln_q_b, H).reshape(1, DIM)
    gseg = jnp.repeat(jnp.eye(H, dtype=f32), D, axis=0) * (1.0 / D)

    full = lambda a: pl.BlockSpec(a.shape, lambda *i: (0,) * a.ndim)

    kvq_shape = jax.ShapeDtypeStruct((B, S, DIM), f32)
    kt, vt, qt = pl.pallas_call(
        _proj_kernel,
        grid=(2, 2, C),
        in_specs=[
            pl.BlockSpec((1, SC, DIM), lambda c, bi, j: (2 * c + bi, j, 0)),
            full(Wk), full(Wv), full(Wq),
            full(ling), full(linb), full(gseg),
            full(gk), full(bk), full(gv), full(bv), full(gq), full(bq),
        ],
        out_specs=[
            pl.BlockSpec((1, SC, DIM), lambda c, bi, j: (2 * c + bi, j, 0)),
        ] * 3,
        out_shape=[kvq_shape] * 3,
        compiler_params=pltpu.CompilerParams(
            dimension_semantics=("parallel", "arbitrary", "arbitrary"),
            vmem_limit_bytes=56 * 1024 * 1024),
        name="titans_proj",
    )(x, Wk, Wv, Wq, ling, linb, gseg, gk, bk, gv, bv, gq, bq)
    kt = kt.reshape(B, S, H, D)
    vt = vt.reshape(B, S, H, D)
    qt = qt.reshape(B, S, H, D)

    mt0 = memory_state.reshape(B * H, D, D).transpose(0, 2, 1)
    mom0 = memory_momentum.reshape(B * H, D, D).transpose(0, 2, 1)
    base_lr = jax.nn.sigmoid(lr_scale) * (LR_BASE * 2.0)
    base_fg = jax.nn.sigmoid(forget_scale) * 0.5
    blr = jnp.tile(jnp.tile(base_lr, B)[:, None], (1, D))
    bfg = jnp.tile(jnp.tile(base_fg, B)[:, None], (1, D))
    w3rep = jnp.repeat(mm_w3, D, axis=1)
    b3rep = jnp.repeat(mm_b3, D).reshape(1, 2 * D)
    b1 = mm_b1.reshape(1, 2 * D)
    b2 = mm_b2.reshape(1, D)

    P = B * H
    outs, mtf, momf = pl.pallas_call(
        _scan_kernel,
        grid=(C,),
        in_specs=[
            pl.BlockSpec(memory_space=pl.ANY),
            pl.BlockSpec(memory_space=pl.ANY),
            pl.BlockSpec(memory_space=pl.ANY),
            full(mt0), full(mom0),
            full(mm_w1), full(b1), full(mm_w2), full(b2),
            full(w3rep), full(b3rep),
            full(blr), full(bfg),
        ],
        out_specs=[
            pl.BlockSpec(memory_space=pl.ANY),
            pl.BlockSpec((P, D, D), lambda j: (0, 0, 0)),
            pl.BlockSpec((P, D, D), lambda j: (0, 0, 0)),
        ],
        out_shape=[
            jax.ShapeDtypeStruct((B, S, H, D), f32),
            jax.ShapeDtypeStruct((P, D, D), f32),
            jax.ShapeDtypeStruct((P, D, D), f32),
        ],
        scratch_shapes=[
            pltpu.VMEM((SC, P, D), f32),
            pltpu.VMEM((SC, P, D), f32),
            pltpu.VMEM((SC, P, D), f32),
            pltpu.VMEM((SC, P, D), f32),
            pltpu.SemaphoreType.DMA((12,)),
            pltpu.SemaphoreType.DMA((4,)),
        ],
        compiler_params=pltpu.CompilerParams(
            dimension_semantics=("arbitrary",),
            vmem_limit_bytes=56 * 1024 * 1024),
        name="titans_scan",
    )(kt, vt, qt, mt0, mom0, mm_w1, b1, mm_w2, b2, w3rep, b3rep, blr, bfg)

    gb = gate_b.reshape(1, DIM)
    outs2 = outs.reshape(B, S, DIM)
    y = pl.pallas_call(
        _out_kernel,
        grid=(2, 2, C),
        in_specs=[
            pl.BlockSpec((1, SC, DIM), lambda c, bi, j: (2 * c + bi, j, 0)),
            pl.BlockSpec((1, SC, DIM), lambda c, bi, j: (2 * c + bi, j, 0)),
            full(Wo), full(gate_w), full(gb), full(ling), full(linb),
        ],
        out_specs=pl.BlockSpec((1, SC, DIM), lambda c, bi, j: (2 * c + bi, j, 0)),
        out_shape=jax.ShapeDtypeStruct((B, S, DIM), f32),
        compiler_params=pltpu.CompilerParams(
            dimension_semantics=("parallel", "arbitrary", "arbitrary"),
            vmem_limit_bytes=56 * 1024 * 1024),
        name="titans_out",
    )(x, outs2, Wo, gate_w, gb, ling, linb)

    m_out = mtf.transpose(0, 2, 1).reshape(B, H, D, D)
    mom_out = momf.transpose(0, 2, 1).reshape(B, H, D, D)
    return y, m_out, mom_out


# speedup vs baseline: 1.0197x; 1.0197x over previous
import functools

import jax
import jax.numpy as jnp
from jax import lax
from jax.experimental import pallas as pl
from jax.experimental.pallas import tpu as pltpu

H = 12
D = 64
DIM = H * D
LR_BASE = 0.1
MOMENTUM = 0.9
SC = 256


def _gelu_exact(x):
    return x * 0.5 * (1.0 + lax.erf(x * 0.7071067811865476))


def _ln_rows(x, g, b, eps=1e-5):
    m = jnp.mean(x, axis=-1, keepdims=True)
    v = jnp.mean((x - m) ** 2, axis=-1, keepdims=True)
    return (x - m) * lax.rsqrt(v + eps) * g + b


def _proj_kernel(x_ref, wk_ref, wv_ref, wq_ref, ling_ref, linb_ref,
                 gseg_ref, gk_ref, bk_ref, gv_ref, bv_ref, gq_ref, bq_ref,
                 kt_ref, vt_ref, qt_ref):
    xn = _ln_rows(x_ref[0], ling_ref[...], linb_ref[...])
    gseg = gseg_ref[...]
    for w_ref, g_ref, b_ref, o_ref in (
        (wk_ref, gk_ref, bk_ref, kt_ref),
        (wv_ref, gv_ref, bv_ref, vt_ref),
        (wq_ref, gq_ref, bq_ref, qt_ref),
    ):
        p = jnp.dot(xn, w_ref[...], preferred_element_type=jnp.float32)
        m = jnp.dot(p, gseg, preferred_element_type=jnp.float32)
        mb = jnp.dot(m, gseg.T, preferred_element_type=jnp.float32) * float(D)
        d = p - mb
        v = jnp.dot(d * d, gseg, preferred_element_type=jnp.float32)
        vb = jnp.dot(v, gseg.T, preferred_element_type=jnp.float32) * float(D)
        o_ref[0] = d * lax.rsqrt(vb + 1e-5) * g_ref[...] + b_ref[...]


def _scan_kernel(kt_hbm, vt_hbm, qt_hbm, m0_ref, mom0_ref,
                 w1_ref, b1_ref, w2_ref, b2_ref, w3_ref, b3_ref,
                 blr_ref, bfg_ref,
                 o_hbm, mt_ref, momt_ref,
                 kbuf, vbuf, qbuf, obuf, insem, outsem):
    j = pl.program_id(0)

    @pl.when(j == 0)
    def _():
        mt_ref[...] = m0_ref[...]
        momt_ref[...] = mom0_ref[...]

    copies = []
    for ai, (src, dst) in enumerate(
            ((kt_hbm, kbuf), (vt_hbm, vbuf), (qt_hbm, qbuf))):
        for bi in range(4):
            cp = pltpu.make_async_copy(
                src.at[bi, pl.ds(j * SC, SC)],
                dst.at[:, bi * H:(bi + 1) * H, :],
                insem.at[ai * 4 + bi])
            cp.start()
            copies.append(cp)
    for cp in copies:
        cp.wait()

    w1 = w1_ref[...]
    b1 = b1_ref[...]
    w2 = w2_ref[...]
    b2 = b2_ref[...]
    w3 = w3_ref[...]
    b3 = b3_ref[...]
    blr = blr_ref[...]
    bfg = bfg_ref[...]

    def step(t, _):
        mt = mt_ref[...]
        mom = momt_ref[...]
        k_t = kbuf[t]
        v_t = vbuf[t]
        q_t = qbuf[t]

        ksub = k_t[:, :, None]
        out_t = jnp.sum(mt * q_t[:, :, None], axis=1)
        pred = jnp.sum(mt * ksub, axis=1)
        surprise = pred - v_t

        cat = jnp.concatenate([k_t, v_t, surprise], axis=1)
        h1 = _gelu_exact(jnp.dot(cat, w1, preferred_element_type=jnp.float32)
                         + b1)
        h2 = _gelu_exact(jnp.dot(h1, w2, preferred_element_type=jnp.float32)
                         + b2)
        mods = jax.nn.sigmoid(jnp.dot(h2, w3, preferred_element_type=jnp.float32)
                              + b3)
        eff_lr = blr * (0.5 + mods[:, :D])
        eff_fg = bfg * (0.5 + mods[:, D:])

        rnorm = 1.0 / (jnp.sqrt(jnp.sum(k_t * k_t, axis=1, keepdims=True)) + 1e-6)
        pred2 = pred * rnorm

        knsub = ksub * rnorm[:, :, None]
        mom2 = MOMENTUM * mom + knsub * surprise[:, None, :]
        fgp2 = eff_fg * pred2
        mt_ref[...] = mt - knsub * fgp2[:, None, :] \
                         - eff_lr[:, None, :] * mom2
        momt_ref[...] = mom2

        obuf[t] = out_t
        return ()

    lax.fori_loop(0, SC, step, (), unroll=2)

    ocopies = []
    for bi in range(4):
        cp = pltpu.make_async_copy(
            obuf.at[:, bi * H:(bi + 1) * H, :],
            o_hbm.at[bi, pl.ds(j * SC, SC)],
            outsem.at[bi])
        cp.start()
        ocopies.append(cp)
    for cp in ocopies:
        cp.wait()


def _out_kernel(x_ref, o_ref, wo_ref, gw_ref, gb_ref, ling_ref, linb_ref,
                y_ref):
    xn = _ln_rows(x_ref[0], ling_ref[...], linb_ref[...])
    out = jnp.dot(o_ref[0], wo_ref[...],
                  preferred_element_type=jnp.float32)
    gate = jax.nn.sigmoid(jnp.dot(xn, gw_ref[...],
                                  preferred_element_type=jnp.float32)
                          + gb_ref[...])
    y_ref[0] = gate * out


@jax.jit
def kernel(x, memory_state, memory_momentum, Wk, Wv, Wq, Wo,
           ln_in_g, ln_in_b, ln_k_g, ln_k_b, ln_v_g, ln_v_b, ln_q_g, ln_q_b,
           mm_w1, mm_b1, mm_w2, mm_b2, mm_w3, mm_b3,
           lr_scale, forget_scale, gate_w, gate_b):
    B, S, _ = x.shape
    C = S // SC
    f32 = jnp.float32

    ling = ln_in_g.reshape(1, DIM)
    linb = ln_in_b.reshape(1, DIM)
    gk, bk = jnp.tile(ln_k_g, H).reshape(1, DIM), jnp.tile(ln_k_b, H).reshape(1, DIM)
    gv, bv = jnp.tile(ln_v_g, H).reshape(1, DIM), jnp.tile(ln_v_b, H).reshape(1, DIM)
    gq, bq = jnp.tile(ln_q_g, H).reshape(1, DIM), jnp.tile(ln_q_b, H).reshape(1, DIM)
    gseg = jnp.repeat(jnp.eye(H, dtype=f32), D, axis=0) * (1.0 / D)

    full = lambda a: pl.BlockSpec(a.shape, lambda *i: (0,) * a.ndim)

    kvq_shape = jax.ShapeDtypeStruct((B, S, DIM), f32)
    kt, vt, qt = pl.pallas_call(
        _proj_kernel,
        grid=(2, 2, C),
        in_specs=[
            pl.BlockSpec((1, SC, DIM), lambda c, bi, j: (2 * c + bi, j, 0)),
            full(Wk), full(Wv), full(Wq),
            full(ling), full(linb), full(gseg),
            full(gk), full(bk), full(gv), full(bv), full(gq), full(bq),
        ],
        out_specs=[
            pl.BlockSpec((1, SC, DIM), lambda c, bi, j: (2 * c + bi, j, 0)),
        ] * 3,
        out_shape=[kvq_shape] * 3,
        compiler_params=pltpu.CompilerParams(
            dimension_semantics=("parallel", "arbitrary", "arbitrary"),
            vmem_limit_bytes=56 * 1024 * 1024),
        name="titans_proj",
    )(x, Wk, Wv, Wq, ling, linb, gseg, gk, bk, gv, bv, gq, bq)
    kt = kt.reshape(B, S, H, D)
    vt = vt.reshape(B, S, H, D)
    qt = qt.reshape(B, S, H, D)

    mt0 = memory_state.reshape(B * H, D, D).transpose(0, 2, 1)
    mom0 = memory_momentum.reshape(B * H, D, D).transpose(0, 2, 1)
    base_lr = jax.nn.sigmoid(lr_scale) * (LR_BASE * 2.0)
    base_fg = jax.nn.sigmoid(forget_scale) * 0.5
    blr = jnp.tile(jnp.tile(base_lr, B)[:, None], (1, D))
    bfg = jnp.tile(jnp.tile(base_fg, B)[:, None], (1, D))
    w3rep = jnp.repeat(mm_w3, D, axis=1)
    b3rep = jnp.repeat(mm_b3, D).reshape(1, 2 * D)
    b1 = mm_b1.reshape(1, 2 * D)
    b2 = mm_b2.reshape(1, D)

    P = B * H
    outs, mtf, momf = pl.pallas_call(
        _scan_kernel,
        grid=(C,),
        in_specs=[
            pl.BlockSpec(memory_space=pl.ANY),
            pl.BlockSpec(memory_space=pl.ANY),
            pl.BlockSpec(memory_space=pl.ANY),
            full(mt0), full(mom0),
            full(mm_w1), full(b1), full(mm_w2), full(b2),
            full(w3rep), full(b3rep),
            full(blr), full(bfg),
        ],
        out_specs=[
            pl.BlockSpec(memory_space=pl.ANY),
            pl.BlockSpec((P, D, D), lambda j: (0, 0, 0)),
            pl.BlockSpec((P, D, D), lambda j: (0, 0, 0)),
        ],
        out_shape=[
            jax.ShapeDtypeStruct((B, S, H, D), f32),
            jax.ShapeDtypeStruct((P, D, D), f32),
            jax.ShapeDtypeStruct((P, D, D), f32),
        ],
        scratch_shapes=[
            pltpu.VMEM((SC, P, D), f32),
            pltpu.VMEM((SC, P, D), f32),
            pltpu.VMEM((SC, P, D), f32),
            pltpu.VMEM((SC, P, D), f32),
            pltpu.SemaphoreType.DMA((12,)),
            pltpu.SemaphoreType.DMA((4,)),
        ],
        compiler_params=pltpu.CompilerParams(
            dimension_semantics=("arbitrary",),
            vmem_limit_bytes=56 * 1024 * 1024),
        name="titans_scan",
    )(kt, vt, qt, mt0, mom0, mm_w1, b1, mm_w2, b2, w3rep, b3rep, blr, bfg)

    gb = gate_b.reshape(1, DIM)
    outs2 = outs.reshape(B, S, DIM)
    y = pl.pallas_call(
        _out_kernel,
        grid=(2, 2, C),
        in_specs=[
            pl.BlockSpec((1, SC, DIM), lambda c, bi, j: (2 * c + bi, j, 0)),
            pl.BlockSpec((1, SC, DIM), lambda c, bi, j: (2 * c + bi, j, 0)),
            full(Wo), full(gate_w), full(gb), full(ling), full(linb),
        ],
        out_specs=pl.BlockSpec((1, SC, DIM), lambda c, bi, j: (2 * c + bi, j, 0)),
        out_shape=jax.ShapeDtypeStruct((B, S, DIM), f32),
        compiler_params=pltpu.CompilerParams(
            dimension_semantics=("parallel", "arbitrary", "arbitrary"),
            vmem_limit_bytes=56 * 1024 * 1024),
        name="titans_out",
    )(x, outs2, Wo, gate_w, gb, ling, linb)

    m_out = mtf.transpose(0, 2, 1).reshape(B, H, D, D)
    mom_out = momf.transpose(0, 2, 1).reshape(B, H, D, D)
    return y, m_out, mom_out
